# Initial kernel scaffold; baseline (speedup 1.0000x reference)
import jax
import jax.numpy as jnp
from jax.experimental import pallas as pl


def kernel(x, edge_index, W_d0a, b_d0a, W_d0b, b_d0b, pool_w, W_d1a, b_d1a, W_d1b, b_d1b, W_u0a, b_u0a, W_u0b, b_u0b):
    raise NotImplementedError("write your pallas kernel here")


# speedup vs baseline: 2.5333x; 2.5333x over previous
import jax
import jax.numpy as jnp
from jax.experimental import pallas as pl

_BJ = 512
_BK = 1024
_BP = 512


def _mmT_kernel(a_ref, z_ref, y_ref):
    k = pl.program_id(1)
    part = jax.lax.dot_general(
        a_ref[...], z_ref[...], (((0,), (0,)), ((), ())),
        preferred_element_type=jnp.float32)

    @pl.when(k == 0)
    def _():
        y_ref[...] = part

    @pl.when(k != 0)
    def _():
        y_ref[...] = y_ref[...] + part


def _mmT(A, Z):
    NP = A.shape[0]
    F = Z.shape[1]
    return pl.pallas_call(
        _mmT_kernel,
        grid=(NP // _BJ, NP // _BK),
        in_specs=[
            pl.BlockSpec((_BK, _BJ), lambda j, k: (k, j)),
            pl.BlockSpec((_BK, F), lambda j, k: (k, 0)),
        ],
        out_specs=pl.BlockSpec((_BJ, F), lambda j, k: (j, 0)),
        out_shape=jax.ShapeDtypeStruct((NP, F), jnp.float32),
    )(A, Z)


def _p0_kernel(arow_ref, acol_ref, c_ref, cs_ref):
    q = pl.program_id(1)
    arow = arow_ref[...]
    acol = acol_ref[...]
    cpart = jnp.sum(jnp.transpose(arow) * acol, axis=0, keepdims=True)
    cspart = jnp.sum(acol, axis=0, keepdims=True)

    @pl.when(q == 0)
    def _():
        c_ref[...] = cpart
        cs_ref[...] = cspart

    @pl.when(q != 0)
    def _():
        c_ref[...] = c_ref[...] + cpart
        cs_ref[...] = cs_ref[...] + cspart


def _p0(A):
    NP = A.shape[0]
    n = NP // _BP
    return pl.pallas_call(
        _p0_kernel,
        grid=(n, n),
        in_specs=[
            pl.BlockSpec((_BP, _BP), lambda p, q: (p, q)),
            pl.BlockSpec((_BP, _BP), lambda p, q: (q, p)),
        ],
        out_specs=[
            pl.BlockSpec((1, _BP), lambda p, q: (0, p)),
            pl.BlockSpec((1, _BP), lambda p, q: (0, p)),
        ],
        out_shape=[
            jax.ShapeDtypeStruct((1, NP), jnp.float32),
            jax.ShapeDtypeStruct((1, NP), jnp.float32),
        ],
    )(A, A)


def _mm_kernel(x_ref, w_ref, o_ref):
    o_ref[...] = jnp.dot(x_ref[...], w_ref[...],
                         preferred_element_type=jnp.float32)


def _mm(X, W):
    NP = X.shape[0]
    return pl.pallas_call(
        _mm_kernel,
        grid=(NP // 1024,),
        in_specs=[
            pl.BlockSpec((1024, X.shape[1]), lambda i: (i, 0)),
            pl.BlockSpec(W.shape, lambda i: (0, 0)),
        ],
        out_specs=pl.BlockSpec((1024, W.shape[1]), lambda i: (i, 0)),
        out_shape=jax.ShapeDtypeStruct((NP, W.shape[1]), jnp.float32),
    )(X, W)


def kernel(x, edge_index, W_d0a, b_d0a, W_d0b, b_d0b, pool_w,
           W_d1a, b_d1a, W_d1b, b_d1b, W_u0a, b_u0a, W_u0b, b_u0b):
    N = x.shape[0]
    NP = ((N + 1023) // 1024) * 1024
    K = N // 2

    src, dst = edge_index[0], edge_index[1]
    w_e = (src != dst).astype(jnp.float32)
    A = jnp.zeros((NP, NP), jnp.float32).at[src, dst].add(w_e)

    rowdot, colsum = _p0(A)
    c = 1.0 + rowdot[0]
    deg0 = colsum[0] + 1.0
    dinv0 = jax.lax.rsqrt(deg0)

    xpad = jnp.pad(x, ((0, NP - N), (0, 0)))

    def edge_conv(h, W, b):
        hw = _mm(h, W)
        agg = _mmT(A, dinv0[:, None] * hw)
        return dinv0[:, None] * agg + (dinv0 * dinv0)[:, None] * hw + b

    h = jax.nn.relu(edge_conv(xpad, W_d0a, b_d0a))
    res0 = jax.nn.relu(edge_conv(h, W_d0b, b_d0b))

    pw_n = pool_w / jnp.linalg.norm(pool_w)
    score_t = jnp.tanh(res0[:N] @ pw_n)
    _, perm = jax.lax.top_k(score_t, K)
    m = jnp.zeros((NP,), jnp.float32).at[perm].set(1.0)
    sc_full = jnp.pad(score_t, (0, NP - N))
    xp = res0 * (m * sc_full)[:, None]

    mB = jnp.broadcast_to(m[:, None], (NP, 128))
    t1 = _mmT(A, mB)[:, 0] + m
    t2 = _mmT(A, jnp.broadcast_to(t1[:, None], (NP, 128)))[:, 0] + t1
    Btm = t2 - c * m
    degp = m * Btm + 1.0
    dinvp = jax.lax.rsqrt(degp)

    def dense_conv(h, W, b):
        hw = _mm(h, W)
        u = (m * dinvp)[:, None] * hw
        v = _mmT(A, u) + u
        Ctu = _mmT(A, v) + v
        Btu = Ctu - c[:, None] * u
        return dinvp[:, None] * (m[:, None] * Btu + dinvp[:, None] * hw) + b

    h = jax.nn.relu(dense_conv(xp, W_d1a, b_d1a))
    h = jax.nn.relu(dense_conv(h, W_d1b, b_d1b))

    xu = res0 + m[:, None] * h

    h = jax.nn.relu(edge_conv(xu, W_u0a, b_u0a))
    out = edge_conv(h, W_u0b, b_u0b)
    return out[:N]
